# Initial kernel scaffold; baseline (speedup 1.0000x reference)
import jax
import jax.numpy as jnp
from jax.experimental import pallas as pl


def kernel(x, edge_index, batch, conv0_w, conv0_b, conv1_w, conv1_b, mlp0_w, mlp0_b, pred_w, pred_b):
    raise NotImplementedError("write your pallas kernel here")


# speedup vs baseline: 9.2193x; 9.2193x over previous
import functools

import jax
import jax.numpy as jnp
from jax import lax
from jax.experimental import pallas as pl
from jax.experimental.pallas import tpu as pltpu
from jax.experimental.pallas import tpu_sc as plsc

NN = 10000
EE = 320000
DD = 128
NG = 8
NCLS = 32
NCORES = 2
NSUB = 16
NWORK = NCORES * NSUB
CH = 80
EPW = EE // NWORK
NCH = EPW // CH
ZR = 624
ZREM = NN - NSUB * ZR
BR = 1000

_MESH = plsc.VectorSubcoreMesh(core_axis_name="c", subcore_axis_name="s")


@functools.partial(
    pl.kernel,
    mesh=_MESH,
    out_type=jax.ShapeDtypeStruct((NCORES, NN, DD), jnp.float32),
    scratch_types=[
        pltpu.VMEM((CH,), jnp.int32),
        pltpu.VMEM((CH, DD), jnp.float32),
        pltpu.VMEM_SHARED((NN, DD), jnp.float32),
    ],
)
def _sc_degree(col_hbm, ones_hbm, zrows_hbm, out_hbm, cidx, onesbuf, degacc):
    c = lax.axis_index("c")
    s = lax.axis_index("s")
    wid = s * NCORES + c
    base = wid * EPW

    pltpu.sync_copy(zrows_hbm, degacc.at[pl.ds(s * ZR, ZR)])

    @pl.when(s == NSUB - 1)
    def _zero_rem():
        pltpu.sync_copy(zrows_hbm.at[pl.ds(0, ZREM)],
                        degacc.at[pl.ds(NSUB * ZR, ZREM)])

    pltpu.sync_copy(ones_hbm, onesbuf)
    plsc.subcore_barrier()

    def body(i, carry):
        pltpu.sync_copy(col_hbm.at[pl.ds(base + i * CH, CH)], cidx)
        pltpu.sync_copy(onesbuf, degacc.at[cidx], add=True)
        return carry

    lax.fori_loop(0, NCH, body, 0)
    plsc.subcore_barrier()
    pltpu.sync_copy(degacc.at[pl.ds(s * ZR, ZR)],
                    out_hbm.at[c, pl.ds(s * ZR, ZR)])

    @pl.when(s == NSUB - 1)
    def _dump_rem():
        pltpu.sync_copy(degacc.at[pl.ds(NSUB * ZR, ZREM)],
                        out_hbm.at[c, pl.ds(NSUB * ZR, ZREM)])


@functools.partial(
    pl.kernel,
    mesh=_MESH,
    out_type=jax.ShapeDtypeStruct((NCORES, NN, DD), jnp.float32),
    scratch_types=[
        pltpu.VMEM((CH,), jnp.int32),
        pltpu.VMEM((CH,), jnp.int32),
        pltpu.VMEM((CH,), jnp.int32),
        pltpu.VMEM((CH,), jnp.int32),
        pltpu.VMEM((CH, DD), jnp.float32),
        pltpu.VMEM((CH, DD), jnp.float32),
        pltpu.VMEM_SHARED((NN, DD), jnp.float32),
        pltpu.SemaphoreType.DMA,
        pltpu.SemaphoreType.DMA,
    ],
)
def _sc_prop(y_hbm, row_hbm, col_hbm, zrows_hbm, out_hbm,
             ridxA, ridxB, cidxA, cidxB, bufA, bufB, acc, semA, semB):
    c = lax.axis_index("c")
    s = lax.axis_index("s")
    wid = s * NCORES + c
    base = wid * EPW

    pltpu.sync_copy(zrows_hbm, acc.at[pl.ds(s * ZR, ZR)])

    @pl.when(s == NSUB - 1)
    def _zero_rem():
        pltpu.sync_copy(zrows_hbm.at[pl.ds(0, ZREM)],
                        acc.at[pl.ds(NSUB * ZR, ZREM)])

    plsc.subcore_barrier()

    pltpu.sync_copy(row_hbm.at[pl.ds(base, CH)], ridxA)
    pltpu.sync_copy(col_hbm.at[pl.ds(base, CH)], cidxA)
    pltpu.async_copy(y_hbm.at[ridxA], bufA, semA)

    def body(g, carry):
        j1 = 2 * g + 1
        pltpu.sync_copy(row_hbm.at[pl.ds(base + j1 * CH, CH)], ridxB)
        pltpu.sync_copy(col_hbm.at[pl.ds(base + j1 * CH, CH)], cidxB)
        pltpu.async_copy(y_hbm.at[ridxB], bufB, semB)

        pltpu.make_async_copy(y_hbm.at[ridxA], bufA, semA).wait()
        pltpu.sync_copy(bufA, acc.at[cidxA], add=True)

        j2 = 2 * g + 2
        pltpu.sync_copy(row_hbm.at[pl.ds(base + j2 * CH, CH)], ridxA)
        pltpu.sync_copy(col_hbm.at[pl.ds(base + j2 * CH, CH)], cidxA)
        pltpu.async_copy(y_hbm.at[ridxA], bufA, semA)

        pltpu.make_async_copy(y_hbm.at[ridxB], bufB, semB).wait()
        pltpu.sync_copy(bufB, acc.at[cidxB], add=True)
        return carry

    lax.fori_loop(0, (NCH - 1) // 2, body, 0)

    pltpu.make_async_copy(y_hbm.at[ridxA], bufA, semA).wait()
    pltpu.sync_copy(bufA, acc.at[cidxA], add=True)

    plsc.subcore_barrier()
    pltpu.sync_copy(acc.at[pl.ds(s * ZR, ZR)],
                    out_hbm.at[c, pl.ds(s * ZR, ZR)])

    @pl.when(s == NSUB - 1)
    def _dump_rem():
        pltpu.sync_copy(acc.at[pl.ds(NSUB * ZR, ZREM)],
                        out_hbm.at[c, pl.ds(NSUB * ZR, ZREM)])


def _dis_body(degp_ref, dis_ref):
    t = degp_ref[...]
    deg = t[0, :, 0] + t[1, :, 0]
    inv = jnp.where(deg > 0.0, lax.rsqrt(jnp.maximum(deg, 1e-12)), 0.0)
    dis_ref[...] = inv[None, :]


def _dis_call(degp):
    return pl.pallas_call(
        _dis_body,
        out_shape=jax.ShapeDtypeStruct((1, NN), jnp.float32),
    )(degp)


def _ta_body(h_ref, dis_ref, w_ref, out_ref, y_ref):
    t = jnp.maximum(h_ref[...], 0.0)
    out_ref[...] = jnp.dot(t, w_ref[...], preferred_element_type=jnp.float32)
    y_ref[...] = t * dis_ref[...]


def _ta_call(h, dis, w):
    grid = NN // BR
    return pl.pallas_call(
        _ta_body,
        grid=(grid,),
        in_specs=[
            pl.BlockSpec((BR, DD), lambda i: (i, 0)),
            pl.BlockSpec((BR, 1), lambda i: (i, 0)),
            pl.BlockSpec((DD, DD), lambda i: (0, 0)),
        ],
        out_specs=[
            pl.BlockSpec((BR, DD), lambda i: (i, 0)),
            pl.BlockSpec((BR, DD), lambda i: (i, 0)),
        ],
        out_shape=[
            jax.ShapeDtypeStruct((NN, DD), jnp.float32),
            jax.ShapeDtypeStruct((NN, DD), jnp.float32),
        ],
    )(h, dis, w)


def _tb_body(p0_ref, p1_ref, dis_ref, w_ref, acc_ref, out_ref, y_ref):
    d = dis_ref[...]
    t = (p0_ref[...] + p1_ref[...]) * d
    out_ref[...] = acc_ref[...] + jnp.dot(
        t, w_ref[...], preferred_element_type=jnp.float32)
    y_ref[...] = t * d


def _tb_call(p0, p1, dis, w, acc):
    grid = NN // BR
    return pl.pallas_call(
        _tb_body,
        grid=(grid,),
        in_specs=[
            pl.BlockSpec((BR, DD), lambda i: (i, 0)),
            pl.BlockSpec((BR, DD), lambda i: (i, 0)),
            pl.BlockSpec((BR, 1), lambda i: (i, 0)),
            pl.BlockSpec((DD, DD), lambda i: (0, 0)),
            pl.BlockSpec((BR, DD), lambda i: (i, 0)),
        ],
        out_specs=[
            pl.BlockSpec((BR, DD), lambda i: (i, 0)),
            pl.BlockSpec((BR, DD), lambda i: (i, 0)),
        ],
        out_shape=[
            jax.ShapeDtypeStruct((NN, DD), jnp.float32),
            jax.ShapeDtypeStruct((NN, DD), jnp.float32),
        ],
    )(p0, p1, dis, w, acc)


def _tl_body(p0_ref, p1_ref, dis_ref, w_ref, b_ref, acc_ref, out_ref):
    t = (p0_ref[...] + p1_ref[...]) * dis_ref[...]
    out_ref[...] = acc_ref[...] + jnp.dot(
        t, w_ref[...], preferred_element_type=jnp.float32) + b_ref[...]


def _tl_call(p0, p1, dis, w, b, acc):
    grid = NN // BR
    return pl.pallas_call(
        _tl_body,
        grid=(grid,),
        in_specs=[
            pl.BlockSpec((BR, DD), lambda i: (i, 0)),
            pl.BlockSpec((BR, DD), lambda i: (i, 0)),
            pl.BlockSpec((BR, 1), lambda i: (i, 0)),
            pl.BlockSpec((DD, DD), lambda i: (0, 0)),
            pl.BlockSpec((1, DD), lambda i: (0, 0)),
            pl.BlockSpec((BR, DD), lambda i: (i, 0)),
        ],
        out_specs=pl.BlockSpec((BR, DD), lambda i: (i, 0)),
        out_shape=jax.ShapeDtypeStruct((NN, DD), jnp.float32),
    )(p0, p1, dis, w, b, acc)


def _pool_body(h_ref, oh_ref, w0_ref, b0_ref, w1_ref, b1_ref, out_ref):
    h = h_ref[...]
    oh = oh_ref[...]
    sums = lax.dot_general(oh, h, (((0,), (0,)), ((), ())),
                           preferred_element_type=jnp.float32)
    cnt = jnp.sum(oh, axis=0)
    mean = sums / jnp.maximum(cnt, 1.0)[:, None]
    mx_rows = []
    for g in range(NG):
        m = oh[:, g:g + 1] > 0.5
        mx_rows.append(jnp.max(jnp.where(m, h, -jnp.inf), axis=0)[None])
    gmax = jnp.concatenate(mx_rows, axis=0)
    gcat = jnp.concatenate([mean, gmax], axis=1)
    gr = jnp.maximum(gcat, 0.0)
    a1 = jnp.maximum(
        jnp.dot(gr, w0_ref[...], preferred_element_type=jnp.float32)
        + b0_ref[...], 0.0)
    out_ref[...] = jnp.dot(
        a1, w1_ref[...], preferred_element_type=jnp.float32) + b1_ref[...]


def _pool_call(h, onehot, w0, b0, w1, b1):
    return pl.pallas_call(
        _pool_body,
        out_shape=jax.ShapeDtypeStruct((NG, NCLS), jnp.float32),
    )(h, onehot, w0, b0, w1, b1)


def kernel(x, edge_index, batch, conv0_w, conv0_b, conv1_w, conv1_b,
           mlp0_w, mlp0_b, pred_w, pred_b):
    row = edge_index[0]
    col = edge_index[1]

    zrows = jnp.zeros((ZR, DD), jnp.float32)
    onesw = jnp.ones((CH, DD), jnp.float32)
    degp = _sc_degree(col, onesw, zrows)
    dis = _dis_call(degp).reshape(NN, 1)

    onehot = (batch[:, None] == jnp.arange(NG, dtype=batch.dtype)
              ).astype(jnp.float32)

    h = x
    for layer in range(2):
        w = conv0_w if layer == 0 else conv1_w
        b = conv0_b if layer == 0 else conv1_b
        out, y = _ta_call(h, dis, w[0])
        for k in (1, 2):
            pp = _sc_prop(y, row, col, zrows)
            out, y = _tb_call(pp[0], pp[1], dis, w[k], out)
        pp = _sc_prop(y, row, col, zrows)
        h = _tl_call(pp[0], pp[1], dis, w[3], b.reshape(1, DD), out)

    return _pool_call(h, onehot, mlp0_w, mlp0_b.reshape(1, 2 * DD),
                      pred_w, pred_b.reshape(1, NCLS))


# speedup vs baseline: 13.8534x; 1.5027x over previous
import functools

import jax
import jax.numpy as jnp
from jax import lax
from jax.experimental import pallas as pl
from jax.experimental.pallas import tpu as pltpu
from jax.experimental.pallas import tpu_sc as plsc

NN = 10000
EE = 320000
DD = 128
NG = 8
NCLS = 32
NCORES = 2
NSUB = 16
NWORK = NCORES * NSUB
CH = 80
EPW = EE // NWORK
NCH = EPW // CH
ZR = 624
ZREM = NN - NSUB * ZR
BR = 1000

_MESH = plsc.VectorSubcoreMesh(core_axis_name="c", subcore_axis_name="s")


CH2 = 40
NCH2 = EPW // CH2
ROT = 5
GRPS = NCH2 // ROT

@functools.partial(
    pl.kernel,
    mesh=_MESH,
    out_type=jax.ShapeDtypeStruct((NCORES, NN, DD), jnp.float32),
    scratch_types=[
        pltpu.VMEM((EPW,), jnp.int32),
    ] + [pltpu.VMEM((CH2,), jnp.int32) for _ in range(ROT)]
      + [pltpu.VMEM((CH2, DD), jnp.float32) for _ in range(ROT)]
      + [pltpu.VMEM_SHARED((NN, DD), jnp.float32)]
      + [pltpu.SemaphoreType.DMA for _ in range(3 * ROT)],
)
def _sc_prop(y_hbm, row_hbm, col_hbm, out_hbm, ridx1d, *rest):
    cbufs = rest[:ROT]
    bufs = rest[ROT:2 * ROT]
    acc = rest[2 * ROT]
    semg = rest[2 * ROT + 1:3 * ROT + 1]
    sems = rest[3 * ROT + 1:4 * ROT + 1]
    semi = rest[4 * ROT + 1:5 * ROT + 1]

    c = lax.axis_index("c")
    s = lax.axis_index("s")
    wid = s * NCORES + c
    base = wid * EPW

    def zrow(i, carry):
        for k in range(DD // 16):
            bufs[0][i, pl.ds(k * 16, 16)] = jnp.zeros((16,), jnp.float32)
        return carry

    lax.fori_loop(0, CH2, zrow, 0)
    for q in range(ZR // CH2):
        pltpu.sync_copy(bufs[0], acc.at[pl.ds(s * ZR + q * CH2, CH2)])
    pltpu.sync_copy(bufs[0].at[pl.ds(0, ZR % CH2)],
                    acc.at[pl.ds(s * ZR + (ZR // CH2) * CH2, ZR % CH2)])

    @pl.when(s == NSUB - 1)
    def _zero_rem():
        pltpu.sync_copy(bufs[0].at[pl.ds(0, ZREM)],
                        acc.at[pl.ds(NSUB * ZR, ZREM)])

    plsc.subcore_barrier()

    pltpu.sync_copy(row_hbm.at[pl.ds(base, EPW)], ridx1d)

    for j0 in range(2):
        pltpu.async_copy(col_hbm.at[pl.ds(base + j0 * CH2, CH2)],
                         cbufs[j0], semi[j0])
        pltpu.async_copy(y_hbm.at[ridx1d.at[pl.ds(j0 * CH2, CH2)]],
                         bufs[j0], semg[j0])

    def grp(g, carry):
        for b in range(ROT):
            j = g * ROT + b
            r2 = (b + 2) % ROT

            @pl.when(j + 2 < NCH2)
            def _prefetch():
                @pl.when(j >= ROT - 2)
                def _drain_scatter():
                    pltpu.make_async_copy(
                        bufs[r2], acc.at[cbufs[r2]], sems[r2]).wait()

                pltpu.async_copy(col_hbm.at[pl.ds(base + (j + 2) * CH2, CH2)],
                                 cbufs[r2], semi[r2])
                pltpu.async_copy(
                    y_hbm.at[ridx1d.at[pl.ds((j + 2) * CH2, CH2)]],
                    bufs[r2], semg[r2])

            pltpu.make_async_copy(
                col_hbm.at[pl.ds(base, CH2)], cbufs[b], semi[b]).wait()
            pltpu.make_async_copy(
                y_hbm.at[ridx1d.at[pl.ds(0, CH2)]], bufs[b], semg[b]).wait()
            pltpu.async_copy(bufs[b], acc.at[cbufs[b]], sems[b], add=True)
        return carry

    lax.fori_loop(0, GRPS, grp, 0)

    for b in range(ROT):
        pltpu.make_async_copy(bufs[b], acc.at[cbufs[b]], sems[b]).wait()

    plsc.subcore_barrier()
    pltpu.sync_copy(acc.at[pl.ds(s * ZR, ZR)],
                    out_hbm.at[c, pl.ds(s * ZR, ZR)])

    @pl.when(s == NSUB - 1)
    def _dump_rem():
        pltpu.sync_copy(acc.at[pl.ds(NSUB * ZR, ZREM)],
                        out_hbm.at[c, pl.ds(NSUB * ZR, ZREM)])


def _dis_body(degp_ref, dis_ref):
    t = degp_ref[...]
    deg = t[0, :, 0] + t[1, :, 0]
    inv = jnp.where(deg > 0.0, lax.rsqrt(jnp.maximum(deg, 1e-12)), 0.0)
    dis_ref[...] = inv[None, :]


def _dis_call(degp):
    return pl.pallas_call(
        _dis_body,
        out_shape=jax.ShapeDtypeStruct((1, NN), jnp.float32),
    )(degp)


def _ta_body(h_ref, dis_ref, w_ref, out_ref, y_ref):
    t = jnp.maximum(h_ref[...], 0.0)
    out_ref[...] = jnp.dot(t, w_ref[...], preferred_element_type=jnp.float32)
    y_ref[...] = t * dis_ref[...]


def _ta_call(h, dis, w):
    grid = NN // BR
    return pl.pallas_call(
        _ta_body,
        grid=(grid,),
        in_specs=[
            pl.BlockSpec((BR, DD), lambda i: (i, 0)),
            pl.BlockSpec((BR, 1), lambda i: (i, 0)),
            pl.BlockSpec((DD, DD), lambda i: (0, 0)),
        ],
        out_specs=[
            pl.BlockSpec((BR, DD), lambda i: (i, 0)),
            pl.BlockSpec((BR, DD), lambda i: (i, 0)),
        ],
        out_shape=[
            jax.ShapeDtypeStruct((NN, DD), jnp.float32),
            jax.ShapeDtypeStruct((NN, DD), jnp.float32),
        ],
    )(h, dis, w)


def _tb_body(p0_ref, p1_ref, dis_ref, w_ref, acc_ref, out_ref, y_ref):
    d = dis_ref[...]
    t = (p0_ref[...] + p1_ref[...]) * d
    out_ref[...] = acc_ref[...] + jnp.dot(
        t, w_ref[...], preferred_element_type=jnp.float32)
    y_ref[...] = t * d


def _tb_call(p0, p1, dis, w, acc):
    grid = NN // BR
    return pl.pallas_call(
        _tb_body,
        grid=(grid,),
        in_specs=[
            pl.BlockSpec((BR, DD), lambda i: (i, 0)),
            pl.BlockSpec((BR, DD), lambda i: (i, 0)),
            pl.BlockSpec((BR, 1), lambda i: (i, 0)),
            pl.BlockSpec((DD, DD), lambda i: (0, 0)),
            pl.BlockSpec((BR, DD), lambda i: (i, 0)),
        ],
        out_specs=[
            pl.BlockSpec((BR, DD), lambda i: (i, 0)),
            pl.BlockSpec((BR, DD), lambda i: (i, 0)),
        ],
        out_shape=[
            jax.ShapeDtypeStruct((NN, DD), jnp.float32),
            jax.ShapeDtypeStruct((NN, DD), jnp.float32),
        ],
    )(p0, p1, dis, w, acc)


def _tl_body(p0_ref, p1_ref, dis_ref, w_ref, b_ref, acc_ref, out_ref):
    t = (p0_ref[...] + p1_ref[...]) * dis_ref[...]
    out_ref[...] = acc_ref[...] + jnp.dot(
        t, w_ref[...], preferred_element_type=jnp.float32) + b_ref[...]


def _tl_call(p0, p1, dis, w, b, acc):
    grid = NN // BR
    return pl.pallas_call(
        _tl_body,
        grid=(grid,),
        in_specs=[
            pl.BlockSpec((BR, DD), lambda i: (i, 0)),
            pl.BlockSpec((BR, DD), lambda i: (i, 0)),
            pl.BlockSpec((BR, 1), lambda i: (i, 0)),
            pl.BlockSpec((DD, DD), lambda i: (0, 0)),
            pl.BlockSpec((1, DD), lambda i: (0, 0)),
            pl.BlockSpec((BR, DD), lambda i: (i, 0)),
        ],
        out_specs=pl.BlockSpec((BR, DD), lambda i: (i, 0)),
        out_shape=jax.ShapeDtypeStruct((NN, DD), jnp.float32),
    )(p0, p1, dis, w, b, acc)


def _pool_body(h_ref, oh_ref, w0_ref, b0_ref, w1_ref, b1_ref, out_ref):
    h = h_ref[...]
    oh = oh_ref[...]
    sums = lax.dot_general(oh, h, (((0,), (0,)), ((), ())),
                           preferred_element_type=jnp.float32)
    cnt = jnp.sum(oh, axis=0)
    mean = sums / jnp.maximum(cnt, 1.0)[:, None]
    mx_rows = []
    for g in range(NG):
        m = oh[:, g:g + 1] > 0.5
        mx_rows.append(jnp.max(jnp.where(m, h, -jnp.inf), axis=0)[None])
    gmax = jnp.concatenate(mx_rows, axis=0)
    gcat = jnp.concatenate([mean, gmax], axis=1)
    gr = jnp.maximum(gcat, 0.0)
    a1 = jnp.maximum(
        jnp.dot(gr, w0_ref[...], preferred_element_type=jnp.float32)
        + b0_ref[...], 0.0)
    out_ref[...] = jnp.dot(
        a1, w1_ref[...], preferred_element_type=jnp.float32) + b1_ref[...]


def _pool_call(h, onehot, w0, b0, w1, b1):
    return pl.pallas_call(
        _pool_body,
        out_shape=jax.ShapeDtypeStruct((NG, NCLS), jnp.float32),
    )(h, onehot, w0, b0, w1, b1)


def kernel(x, edge_index, batch, conv0_w, conv0_b, conv1_w, conv1_b,
           mlp0_w, mlp0_b, pred_w, pred_b):
    row = edge_index[0]
    col = edge_index[1]

    ones_tab = jnp.ones((NN, DD), jnp.float32)
    rows_lin = jnp.tile(jnp.arange(EPW, dtype=jnp.int32), NWORK)
    degp = _sc_prop(ones_tab, rows_lin, col)
    dis = _dis_call(degp).reshape(NN, 1)

    onehot = (batch[:, None] == jnp.arange(NG, dtype=batch.dtype)
              ).astype(jnp.float32)

    h = x
    for layer in range(2):
        w = conv0_w if layer == 0 else conv1_w
        b = conv0_b if layer == 0 else conv1_b
        out, y = _ta_call(h, dis, w[0])
        for k in (1, 2):
            pp = _sc_prop(y, row, col)
            out, y = _tb_call(pp[0], pp[1], dis, w[k], out)
        pp = _sc_prop(y, row, col)
        h = _tl_call(pp[0], pp[1], dis, w[3], b.reshape(1, DD), out)

    return _pool_call(h, onehot, mlp0_w, mlp0_b.reshape(1, 2 * DD),
                      pred_w, pred_b.reshape(1, NCLS))
